# Initial kernel scaffold; baseline (speedup 1.0000x reference)
import jax
import jax.numpy as jnp
from jax.experimental import pallas as pl


def kernel(x_audio, x_text, x_visual, ei_audio_past, ei_visual_past, ei_text_past, ei_audio_future, ei_visual_future, ei_text_future, ei_audio_self, ei_visual_self, ei_text_self, ei_audio_cross_visual, ei_audio_cross_text, ei_visual_cross_audio, ei_visual_cross_text, ei_text_cross_audio, ei_text_cross_visual, params):
    raise NotImplementedError("write your pallas kernel here")


# speedup vs baseline: 6.2314x; 6.2314x over previous
import functools
import numpy as np
import jax
import jax.numpy as jnp
from jax import lax
from jax.experimental import pallas as pl
from jax.experimental.pallas import tpu as pltpu, tpu_sc as plsc

N = 10000
E = 20000
D = 128
H = 2
NPAD = 10240
NH = 30000
NHP = 30720
HALF = 7552
SCR = 2 * HALF
L = 16
NC = 2
NS = 16
NW = NC * NS
CH = 512
CHK = 256
ADEG = 9 * NPAD
ASEG = 2 * NHP
EPS_FA = 0.1

_mesh = plsc.VectorSubcoreMesh(core_axis_name="c", subcore_axis_name="s")
_f32 = jnp.float32
_i32 = jnp.int32


def _sds(shape, dtype=_f32):
    return jax.ShapeDtypeStruct(shape, dtype)


def _zero_rows(rows_v, nrows):
    def zr(r, _):
        for j in range(D // L):
            rows_v[r, pl.ds(j * L, L)] = jnp.zeros((L,), _f32)
        return 0
    lax.fori_loop(0, nrows, zr, 0)


def _bsum(v):
    dn = lax.GatherDimensionNumbers(offset_dims=(),
                                    collapsed_slice_dims=(0,),
                                    start_index_map=(0,))
    lane = lax.broadcasted_iota(_i32, (L,), 0)
    for sh in (8, 4, 2, 1):
        perm = (lane ^ sh).reshape(L, 1)
        v = v + lax.gather(v, perm, dn, (1,),
                           mode=lax.GatherScatterMode.PROMISE_IN_BOUNDS)
    return v


def _chunked_copies(total):
    out = []
    off = 0
    while off < total:
        sz = min(CH, total - off)
        out.append((off, sz))
        off += sz
    return out


@functools.lru_cache(maxsize=None)
def _make_k2(ec, mode):
    ew = ec // NS
    nch = ew // CH
    zr = HALF // NS
    apad = HALF + 8

    scratch = [
        pltpu.VMEM((CH,), _i32),
        pltpu.VMEM((CH,), _i32),
        pltpu.VMEM((CH, D), _f32),
        pltpu.VMEM((CH,), _f32),
    ]
    if mode == "fa":
        scratch += [pltpu.VMEM((CH,), _i32), pltpu.VMEM((CH,), _i32),
                    pltpu.VMEM((CH,), _f32), pltpu.VMEM((CH,), _f32),
                    pltpu.VMEM((CH,), _f32), pltpu.VMEM((CH,), _f32)]
    elif mode == "attn":
        scratch += [pltpu.VMEM((CH,), _i32), pltpu.VMEM((CH,), _f32)]
    scratch += [pltpu.VMEM_SHARED((apad, D), _f32), pltpu.SemaphoreType.DMA]

    def body(*refs):
        if mode == "fa":
            (table, gidx_h, sidx_h, dgl_h, sgl_h, disT, alT, arT, out_ref,
             gidx_v, sidx_v, rows_v, c_v, dg_v, sg_v, dd_v, dsv_v, ad_v,
             ar_v, acc, sem) = refs
        elif mode == "attn":
            (table, gidx_h, sidx_h, cv_h, hx_h, sinvT, out_ref,
             gidx_v, sidx_v, rows_v, c_v, hx_v, sv_v, acc, sem) = refs
        else:
            (table, gidx_h, sidx_h, out_ref,
             gidx_v, sidx_v, rows_v, c_v, acc, sem) = refs
        c = lax.axis_index("c")
        s = lax.axis_index("s")
        _zero_rows(rows_v, CH)
        base_r = s * zr
        for off, sz in _chunked_copies(zr):
            pltpu.sync_copy(rows_v.at[pl.ds(0, sz)],
                            acc.at[pl.ds(base_r + off, sz)])
        plsc.subcore_barrier()

        def chunk(ch, _):
            off = s * ew + ch * CH
            pltpu.sync_copy(gidx_h.at[pl.ds(off, CH)], gidx_v)
            pltpu.sync_copy(sidx_h.at[c, pl.ds(off, CH)], sidx_v)
            pltpu.async_copy(table.at[gidx_v], rows_v, sem).wait()
            if mode == "fa":
                pltpu.sync_copy(dgl_h.at[pl.ds(off, CH)], dg_v)
                pltpu.sync_copy(sgl_h.at[pl.ds(off, CH)], sg_v)
                pltpu.async_copy(disT.at[dg_v], dd_v, sem).wait()
                pltpu.async_copy(disT.at[sg_v], dsv_v, sem).wait()
                pltpu.async_copy(alT.at[dg_v], ad_v, sem).wait()
                pltpu.async_copy(arT.at[sg_v], ar_v, sem).wait()

                def cgrp(i, __):
                    sl = pl.ds(i * L, L)
                    z = ad_v[sl] + ar_v[sl]
                    ez = jnp.exp(-2.0 * jnp.abs(z))
                    th = (1.0 - ez) / (1.0 + ez) * jnp.sign(z)
                    c_v[sl] = dd_v[sl] * dsv_v[sl] * th
                    return 0
                lax.fori_loop(0, CH // L, cgrp, 0)
            elif mode == "attn":
                pltpu.sync_copy(cv_h.at[pl.ds(off, CH)], c_v)
                pltpu.sync_copy(hx_h.at[pl.ds(off, CH)], hx_v)
                pltpu.async_copy(sinvT.at[hx_v], sv_v, sem).wait()

                def agrp(i, __):
                    sl = pl.ds(i * L, L)
                    c_v[sl] = c_v[sl] * sv_v[sl]
                    return 0
                lax.fori_loop(0, CH // L, agrp, 0)
            if mode != "plain":
                def wgrp(g, __):
                    cvec = c_v[pl.ds(g * L, L)]
                    for j in range(L):
                        e = g * L + j
                        w16 = jnp.full((L,), cvec[j], _f32)
                        for q in range(D // L):
                            sl = pl.ds(q * L, L)
                            rows_v[e, sl] = rows_v[e, sl] * w16
                    return 0
                lax.fori_loop(0, CH // L, wgrp, 0)
            pltpu.sync_copy(rows_v, acc.at[sidx_v], add=True)
            return 0
        lax.fori_loop(0, nch, chunk, 0)
        plsc.subcore_barrier()
        for off, sz in _chunked_copies(zr):
            pltpu.sync_copy(acc.at[pl.ds(base_r + off, sz)],
                            out_ref.at[c, pl.ds(base_r + off, sz)])

    return pl.kernel(body, out_type=_sds((NC, HALF, D)), mesh=_mesh,
                     scratch_types=scratch)


@functools.lru_cache(maxsize=None)
def _make_k1(ew, aused, const1):
    nch = ew // CH
    zr = aused // NS
    scratch = [
        pltpu.VMEM((CH,), _i32),
        pltpu.VMEM((CH,), _f32),
        pltpu.VMEM_SHARED((aused + 8,), _f32),
        pltpu.SemaphoreType.DMA,
    ]

    def body(*refs):
        if const1:
            sidx_h, out_ref, sidx_v, vals_v, acc, sem = refs
        else:
            sidx_h, vals_h, out_ref, sidx_v, vals_v, acc, sem = refs
        c = lax.axis_index("c")
        s = lax.axis_index("s")
        r = c * NS + s

        def fill(i, _):
            v = jnp.ones((L,), _f32) if const1 else jnp.zeros((L,), _f32)
            vals_v[pl.ds(i * L, L)] = v
            return 0
        lax.fori_loop(0, CH // L, fill, 0)

        if const1:
            zsrc = sidx_v
        def zfill(i, _):
            vals_v[pl.ds(i * L, L)] = jnp.zeros((L,), _f32)
            return 0
        lax.fori_loop(0, CH // L, zfill, 0)
        for off, sz in _chunked_copies(zr):
            pltpu.sync_copy(vals_v.at[pl.ds(0, sz)],
                            acc.at[pl.ds(s * zr + off, sz)])
        if const1:
            def refill(i, _):
                vals_v[pl.ds(i * L, L)] = jnp.ones((L,), _f32)
                return 0
            lax.fori_loop(0, CH // L, refill, 0)
        plsc.subcore_barrier()

        def chunk(ch, _):
            off = ch * CH
            pltpu.sync_copy(sidx_h.at[r, pl.ds(off, CH)], sidx_v)
            if not const1:
                pltpu.sync_copy(vals_h.at[r, pl.ds(off, CH)], vals_v)
            pltpu.sync_copy(vals_v, acc.at[sidx_v], add=True)
            return 0
        lax.fori_loop(0, nch, chunk, 0)
        plsc.subcore_barrier()
        for off, sz in _chunked_copies(zr):
            pltpu.sync_copy(acc.at[pl.ds(s * zr + off, sz)],
                            out_ref.at[c, pl.ds(s * zr + off, sz)])

    return pl.kernel(body, out_type=_sds((NC, aused)), mesh=_mesh,
                     scratch_types=scratch)


@functools.lru_cache(maxsize=None)
def _make_k3(ew):
    nch = ew // CHK
    scratch = [
        pltpu.VMEM((CHK,), _i32),
        pltpu.VMEM((CHK,), _i32),
        pltpu.VMEM((CHK, D), _f32),
        pltpu.VMEM((CHK, D), _f32),
        pltpu.VMEM((CHK,), _f32),
        pltpu.SemaphoreType.DMA,
    ]

    def body(qT, kT, qidx_h, kidx_h, out_ref,
             qidx_v, kidx_v, qr_v, kr_v, lg_v, sem):
        c = lax.axis_index("c")
        s = lax.axis_index("s")
        r = c * NS + s
        lane = lax.broadcasted_iota(_i32, (L,), 0)

        def chunk(ch, _):
            off = ch * CHK
            pltpu.sync_copy(qidx_h.at[r, pl.ds(off, CHK)], qidx_v)
            pltpu.sync_copy(kidx_h.at[r, pl.ds(off, CHK)], kidx_v)
            pltpu.async_copy(qT.at[qidx_v], qr_v, sem).wait()
            pltpu.async_copy(kT.at[kidx_v], kr_v, sem).wait()

            def grp(g, __):
                res = jnp.zeros((L,), _f32)
                for j in range(L):
                    e = g * L + j
                    acc = qr_v[e, pl.ds(0, L)] * kr_v[e, pl.ds(0, L)]
                    for q in range(1, D // L):
                        sl = pl.ds(q * L, L)
                        acc = acc + qr_v[e, sl] * kr_v[e, sl]
                    v = _bsum(acc)
                    res = jnp.where(lane == j, v, res)
                lg_v[pl.ds(g * L, L)] = res
                return 0
            lax.fori_loop(0, CHK // L, grp, 0)
            pltpu.sync_copy(lg_v, out_ref.at[r, pl.ds(off, CHK)])
            return 0
        lax.fori_loop(0, nch, chunk, 0)

    return pl.kernel(body, out_type=_sds((NW, ew)), mesh=_mesh,
                     scratch_types=scratch)


def _tc_dis(degp):
    def body(d_ref, o_ref):
        deg = d_ref[0] + d_ref[1]
        dis = jnp.where(deg > 0.0, lax.rsqrt(jnp.maximum(deg, 1.0)), 0.0)
        o_ref[...] = dis
    return pl.pallas_call(body, out_shape=_sds((ADEG // D, D)))(degp)


def _tc_alar(x_all, W):
    bn = 1024
    nt = NPAD // bn

    def body(x_ref, w_ref, o_ref):
        o_ref[0] = jnp.dot(x_ref[0], w_ref[0],
                           preferred_element_type=_f32)
    return pl.pallas_call(
        body,
        grid=(3, nt),
        in_specs=[pl.BlockSpec((1, bn, D), lambda t, j: (t, j, 0)),
                  pl.BlockSpec((1, D, D), lambda t, j: (t, 0, 0))],
        out_specs=pl.BlockSpec((1, bn, D), lambda t, j: (t, j, 0)),
        out_shape=_sds((3, NPAD, D)),
    )(x_all, W)


def _tc_combine(x_all, fa, gA, gB, wA, wB, wR, bS):
    bn = 1024
    nt = NPAD // bn

    def body(x_ref, fa_ref, ga_ref, gb_ref, wa_ref, wb_ref, wr_ref, bs_ref,
             y_ref, s1_ref, s2_ref):
        j = pl.program_id(1)
        xv = x_ref[0]
        yv = (fa_ref[0] + EPS_FA * 3.0 * xv
              + jnp.dot(ga_ref[0], wa_ref[0], preferred_element_type=_f32)
              + jnp.dot(gb_ref[0], wb_ref[0], preferred_element_type=_f32)
              + jnp.dot(xv, wr_ref[0], preferred_element_type=_f32)
              + bs_ref[0, 0]) * 0.2
        rows = lax.broadcasted_iota(_i32, (bn, D), 0) + j * bn
        yv = jnp.where(rows < N, yv, 0.0)
        y_ref[0] = yv

        @pl.when(j == 0)
        def _():
            s1_ref[...] = jnp.zeros_like(s1_ref)
            s2_ref[...] = jnp.zeros_like(s2_ref)
        s1_ref[...] += jnp.sum(yv, axis=0)[None, None]
        s2_ref[...] += jnp.sum(yv * yv, axis=0)[None, None]

    return pl.pallas_call(
        body,
        grid=(3, nt),
        in_specs=[pl.BlockSpec((1, bn, D), lambda t, j: (t, j, 0)),
                  pl.BlockSpec((1, bn, D), lambda t, j: (t, j, 0)),
                  pl.BlockSpec((1, bn, D), lambda t, j: (t, j, 0)),
                  pl.BlockSpec((1, bn, D), lambda t, j: (t, j, 0)),
                  pl.BlockSpec((1, D, D), lambda t, j: (t, 0, 0)),
                  pl.BlockSpec((1, D, D), lambda t, j: (t, 0, 0)),
                  pl.BlockSpec((1, D, D), lambda t, j: (t, 0, 0)),
                  pl.BlockSpec((1, 1, D), lambda t, j: (t, 0, 0))],
        out_specs=[pl.BlockSpec((1, bn, D), lambda t, j: (t, j, 0)),
                   pl.BlockSpec((1, 1, D), lambda t, j: (t, 0, 0)),
                   pl.BlockSpec((1, 1, D), lambda t, j: (t, 0, 0))],
        out_shape=[_sds((3, NPAD, D)), _sds((3, 1, D)), _sds((3, 1, D))],
    )(x_all, fa, gA, gB, wA, wB, wR, bS.reshape(3, 1, D))


def _tc_pairnorm(y, s1, s2):
    bn = 1024
    nt = NPAD // bn

    def body(y_ref, s1_ref, s2_ref, o_ref):
        mu = s1_ref[0, 0] / float(N)
        tot = jnp.sum(s2_ref[0, 0]) - float(N) * jnp.sum(mu * mu)
        scale = 1.0 / jnp.sqrt(1e-6 + tot / float(N))
        o_ref[0] = jnp.maximum((y_ref[0] - mu) * scale, 0.0)

    return pl.pallas_call(
        body,
        grid=(3, nt),
        in_specs=[pl.BlockSpec((1, bn, D), lambda t, j: (t, j, 0)),
                  pl.BlockSpec((1, 1, D), lambda t, j: (t, 0, 0)),
                  pl.BlockSpec((1, 1, D), lambda t, j: (t, 0, 0))],
        out_specs=pl.BlockSpec((1, bn, D), lambda t, j: (t, j, 0)),
        out_shape=_sds((3, NPAD, D)),
    )(y, s1, s2)


def _tc_qkv(hx, wq, bq, wk, bk, wv, bv, ws, bs):
    bn = 1024
    nt = NHP // bn

    def body(x_ref, wq_ref, bq_ref, wk_ref, bk_ref, wv_ref, bv_ref,
             ws_ref, bs_ref, q_ref, k_ref, v_ref, sk_ref):
        xv = x_ref[...]
        q_ref[...] = jnp.dot(xv, wq_ref[...],
                             preferred_element_type=_f32) + bq_ref[0]
        k_ref[...] = jnp.dot(xv, wk_ref[...],
                             preferred_element_type=_f32) + bk_ref[0]
        v_ref[...] = jnp.dot(xv, wv_ref[...],
                             preferred_element_type=_f32) + bv_ref[0]
        sk_ref[...] = jnp.dot(xv, ws_ref[...],
                              preferred_element_type=_f32) + bs_ref[0]

    hd = H * D
    return pl.pallas_call(
        body,
        grid=(nt,),
        in_specs=[pl.BlockSpec((bn, D), lambda j: (j, 0)),
                  pl.BlockSpec((D, hd), lambda j: (0, 0)),
                  pl.BlockSpec((1, hd), lambda j: (0, 0)),
                  pl.BlockSpec((D, hd), lambda j: (0, 0)),
                  pl.BlockSpec((1, hd), lambda j: (0, 0)),
                  pl.BlockSpec((D, hd), lambda j: (0, 0)),
                  pl.BlockSpec((1, hd), lambda j: (0, 0)),
                  pl.BlockSpec((D, D), lambda j: (0, 0)),
                  pl.BlockSpec((1, D), lambda j: (0, 0))],
        out_specs=[pl.BlockSpec((bn, hd), lambda j: (j, 0)),
                   pl.BlockSpec((bn, hd), lambda j: (j, 0)),
                   pl.BlockSpec((bn, hd), lambda j: (j, 0)),
                   pl.BlockSpec((bn, D), lambda j: (j, 0))],
        out_shape=[_sds((NHP, hd)), _sds((NHP, hd)), _sds((NHP, hd)),
                   _sds((NHP, D))],
    )(hx, wq, bq, wk, bk, wv, bv, ws, bs)


def _tc_max(lgf):
    def body(l_ref, m_ref):
        m_ref[...] = jnp.full((8, D), jnp.max(l_ref[...]), _f32)
    return pl.pallas_call(body, out_shape=_sds((8, D)))(lgf)


def _tc_exp(lgf, m):
    inv = 1.0 / float(np.sqrt(D))

    def body(l_ref, m_ref, e_ref):
        e_ref[...] = jnp.exp((l_ref[...] - m_ref[0, 0]) * inv)
    return pl.pallas_call(body, out_shape=_sds(lgf.shape))(lgf, m)


def _tc_sinv(sparts):
    def body(p_ref, o_ref):
        sv = p_ref[0] + p_ref[1]
        o_ref[...] = 1.0 / (sv + 1e-16)
    return pl.pallas_call(body, out_shape=_sds((ASEG // D, D)))(sparts)


def _tc_attn_out(o0, o1, skip):
    bn = 1024
    nt = NHP // bn

    def body(o0_ref, o1_ref, sk_ref, h_ref, s1_ref, s2_ref):
        j = pl.program_id(0)
        hv = (o0_ref[...] + o1_ref[...]) * 0.5 + sk_ref[...]
        rows = lax.broadcasted_iota(_i32, (bn, D), 0) + j * bn
        hv = jnp.where(rows < NH, hv, 0.0)
        h_ref[...] = hv

        @pl.when(j == 0)
        def _():
            s1_ref[...] = jnp.zeros_like(s1_ref)
            s2_ref[...] = jnp.zeros_like(s2_ref)
        s1_ref[...] += jnp.sum(hv, axis=0, keepdims=True)
        s2_ref[...] += jnp.sum(hv * hv, axis=0, keepdims=True)

    return pl.pallas_call(
        body,
        grid=(nt,),
        in_specs=[pl.BlockSpec((bn, D), lambda j: (j, 0)),
                  pl.BlockSpec((bn, D), lambda j: (j, 0)),
                  pl.BlockSpec((bn, D), lambda j: (j, 0))],
        out_specs=[pl.BlockSpec((bn, D), lambda j: (j, 0)),
                   pl.BlockSpec((1, D), lambda j: (0, 0)),
                   pl.BlockSpec((1, D), lambda j: (0, 0))],
        out_shape=[_sds((NHP, D)), _sds((1, D)), _sds((1, D))],
    )(o0, o1, skip)


def _tc_bn(hp, s1, s2, gamma, beta):
    bn = 1024
    nt = NHP // bn

    def body(h_ref, s1_ref, s2_ref, g_ref, b_ref, o_ref):
        mu = s1_ref[0] / float(NH)
        var = s2_ref[0] / float(NH) - mu * mu
        xn = g_ref[0] * (h_ref[...] - mu) / jnp.sqrt(var + 1e-5) + b_ref[0]
        o_ref[...] = jnp.where(xn > 0.0, xn, 0.01 * xn)

    return pl.pallas_call(
        body,
        grid=(nt,),
        in_specs=[pl.BlockSpec((bn, D), lambda j: (j, 0)),
                  pl.BlockSpec((1, D), lambda j: (0, 0)),
                  pl.BlockSpec((1, D), lambda j: (0, 0)),
                  pl.BlockSpec((1, D), lambda j: (0, 0)),
                  pl.BlockSpec((1, D), lambda j: (0, 0))],
        out_specs=pl.BlockSpec((bn, D), lambda j: (j, 0)),
        out_shape=_sds((NHP, D)),
    )(hp, s1, s2, gamma, beta)


def _lay(a, nr, ewp, fill):
    per = a.shape[0] // nr
    a = a.reshape(nr, per)
    return jnp.pad(a, ((0, 0), (0, ewp - per)), constant_values=fill)


def _lay_flat(a, nr, ewp, fill):
    return _lay(a, nr, ewp, fill).reshape(-1)


def _range_sidx(gglobal, base, fill_mask=None):
    loc = gglobal - base
    ok = (gglobal >= base) & (gglobal < base + HALF)
    if fill_mask is not None:
        ok = ok & fill_mask
    return jnp.where(ok, loc, HALF).astype(_i32)


def kernel(x_audio, x_text, x_visual,
           ei_audio_past, ei_visual_past, ei_text_past,
           ei_audio_future, ei_visual_future, ei_text_future,
           ei_audio_self, ei_visual_self, ei_text_self,
           ei_audio_cross_visual, ei_audio_cross_text,
           ei_visual_cross_audio, ei_visual_cross_text,
           ei_text_cross_audio, ei_text_cross_visual,
           params):
    fa_types = [('audio', 'past'), ('visual', 'past'), ('text', 'past'),
                ('audio', 'future'), ('visual', 'future'),
                ('text', 'future'),
                ('audio', 'self'), ('visual', 'self'), ('text', 'self')]
    gc_types = [('audio', 'visual'), ('audio', 'text'),
                ('visual', 'audio'), ('visual', 'text'),
                ('text', 'audio'), ('text', 'visual')]
    tmap = {'audio': 0, 'text': 1, 'visual': 2}
    fa_ei = [ei_audio_past, ei_visual_past, ei_text_past,
             ei_audio_future, ei_visual_future, ei_text_future,
             ei_audio_self, ei_visual_self, ei_text_self]
    gc_ei = [ei_audio_cross_visual, ei_audio_cross_text,
             ei_visual_cross_audio, ei_visual_cross_text,
             ei_text_cross_audio, ei_text_cross_visual]

    fa_src = [e[0].astype(_i32) for e in fa_ei]
    fa_dst = [e[1].astype(_i32) for e in fa_ei]
    gc_src = [e[0].astype(_i32) for e in gc_ei]
    gc_dst = [e[1].astype(_i32) for e in gc_ei]

    deg_sidx = jnp.concatenate(
        [fa_dst[i] + i * NPAD for i in range(9)])
    deg_sidx = _lay(deg_sidx, NW, 11 * CH, ADEG)

    ec_fa = 16 * 11264
    fa_g = jnp.concatenate(
        [fa_src[i] + tmap[t] * NPAD for i, (t, _) in enumerate(fa_types)])
    fa_gidx = _lay_flat(fa_g, NS, 11264, 0)
    fa_sg = jnp.concatenate(
        [fa_dst[i] + tmap[t] * N for i, (t, _) in enumerate(fa_types)])
    fa_sg_l = _lay_flat(fa_sg, NS, 11264, -1)
    fa_mask = fa_sg_l >= 0
    fa_sidx = [jnp.stack([_range_sidx(fa_sg_l, j * SCR + c * HALF, fa_mask)
                          for c in range(NC)]) for j in range(2)]
    fa_dgl = _lay_flat(jnp.concatenate(
        [fa_dst[i] + i * NPAD for i in range(9)]), NS, 11264, 0)
    fa_sgl = _lay_flat(jnp.concatenate(
        [fa_src[i] + i * NPAD for i in range(9)]), NS, 11264, 0)

    ec_gc = 16 * 7680
    gc_g = jnp.concatenate(
        [gc_src[g] + tmap[s] * NPAD for g, (s, _) in enumerate(gc_types)])
    gc_gidx = _lay_flat(gc_g, NS, 7680, 0)
    gc_sg = jnp.concatenate(
        [gc_dst[g] + g * N for g in range(6)])
    gc_sg_l = _lay_flat(gc_sg, NS, 7680, -1)
    gc_mask = gc_sg_l >= 0
    gc_sidx = [jnp.stack([_range_sidx(gc_sg_l, j * SCR + c * HALF, gc_mask)
                          for c in range(NC)]) for j in range(4)]

    hsrc = jnp.concatenate(
        [fa_src[i] + tmap[t] * N for i, (t, _) in enumerate(fa_types)]
        + [gc_src[g] + tmap[s] * N for g, (s, _) in enumerate(gc_types)])
    hdst = jnp.concatenate(
        [fa_dst[i] + tmap[t] * N for i, (t, _) in enumerate(fa_types)]
        + [gc_dst[g] + tmap[t] * N for g, (_, t) in enumerate(gc_types)])

    ew_lg = 18944
    qp = jnp.stack([hdst * 2, hdst * 2 + 1], axis=1).reshape(-1)
    kp = jnp.stack([hsrc * 2, hsrc * 2 + 1], axis=1).reshape(-1)
    q_idx = _lay(qp, NW, ew_lg, 0)
    k_idx = _lay(kp, NW, ew_lg, 0)

    sp = jnp.stack([hdst, hdst + NHP], axis=1).reshape(-1)
    seg_sidx = _lay(sp, NW, ew_lg, ASEG)

    ec_t3 = 16 * 18944
    t3_gidx = _lay_flat(hsrc, NS, 18944, 0)
    t3_hd = _lay_flat(hdst, NS, 18944, -1)
    t3_mask = t3_hd >= 0
    t3_sidx = [jnp.stack([_range_sidx(t3_hd, j * SCR + c * HALF, t3_mask)
                          for c in range(NC)]) for j in range(2)]
    t3_hidx = [jnp.where(t3_mask, t3_hd + h * NHP, 0).astype(_i32)
               for h in range(H)]

    fav = {'audio': 0, 'visual': 1, 'text': 2}
    ga_of = [2, 1, 0]
    gb_of = [4, 3, 5]

    k1_deg = _make_k1(11 * CH, ADEG, True)
    degp = k1_deg(deg_sidx)
    dis = _tc_dis(degp.reshape(NC, ADEG // D, D)).reshape(-1)

    def pad_rows(x):
        return jnp.pad(x, ((0, NPAD - N), (0, 0)))

    x_all = jnp.stack([pad_rows(x_audio), pad_rows(x_text),
                       pad_rows(x_visual)])

    k2_fa = _make_k2(ec_fa, "fa")
    k2_gc = _make_k2(ec_gc, "plain")

    for lp in params['layers']:
        Wl = []
        for t in ('audio', 'text', 'visual'):
            cols = []
            for r in range(3):
                i = 3 * r + fav[t]
                cols.append(lp['fa'][i]['att_l'])
                cols.append(lp['fa'][i]['att_r'])
            Wt = jnp.stack(cols, axis=1)
            Wl.append(jnp.pad(Wt, ((0, 0), (0, D - 6))))
        alar = _tc_alar(x_all, jnp.stack(Wl))
        alT = jnp.concatenate(
            [alar[tmap[t], :, 2 * (i // 3)] for i, (t, _) in
             enumerate(fa_types)])
        arT = jnp.concatenate(
            [alar[tmap[t], :, 2 * (i // 3) + 1] for i, (t, _) in
             enumerate(fa_types)])

        xtab = x_all.reshape(3 * NPAD, D)
        fafull = jnp.concatenate(
            [k2_fa(xtab, fa_gidx, fa_sidx[j], fa_dgl, fa_sgl,
                   dis, alT, arT).reshape(SCR, D) for j in range(2)])
        fa_in = jnp.stack([pad_rows(fafull[t * N:(t + 1) * N])
                           for t in range(3)])

        gcfull = jnp.concatenate(
            [k2_gc(xtab, gc_gidx, gc_sidx[j]).reshape(SCR, D)
             for j in range(4)])
        gA = jnp.stack([pad_rows(
            gcfull[ga_of[t] * N:(ga_of[t] + 1) * N]) for t in range(3)])
        gB = jnp.stack([pad_rows(
            gcfull[gb_of[t] * N:(gb_of[t] + 1) * N]) for t in range(3)])
        wA = jnp.stack([lp['gc'][ga_of[t]]['W_rel'] for t in range(3)])
        wB = jnp.stack([lp['gc'][gb_of[t]]['W_rel'] for t in range(3)])
        wR = jnp.stack([lp['gc'][ga_of[t]]['W_root']
                        + lp['gc'][gb_of[t]]['W_root'] for t in range(3)])
        bS = jnp.stack([lp['gc'][ga_of[t]]['b'] + lp['gc'][gb_of[t]]['b']
                        for t in range(3)])

        y, s1, s2 = _tc_combine(x_all, fa_in, gA, gB, wA, wB, wR, bS)
        x_all = _tc_pairnorm(y, s1, s2)

    tr = params['trans']
    hx = jnp.concatenate([x_all[0, :N], x_all[1, :N], x_all[2, :N]])
    hx = jnp.pad(hx, ((0, NHP - NH), (0, 0)))
    q, k, v, skip = _tc_qkv(
        hx, tr['Wq'], tr['bq'].reshape(1, -1), tr['Wk'],
        tr['bk'].reshape(1, -1), tr['Wv'], tr['bv'].reshape(1, -1),
        tr['Wskip'], tr['bskip'].reshape(1, -1))
    qT = q.reshape(NHP * H, D)
    kT = k.reshape(NHP * H, D)
    vh = v.reshape(NHP, H, D).transpose(1, 0, 2)

    k3 = _make_k3(ew_lg)
    lg = k3(qT, kT, q_idx, k_idx)
    lgf = lg.reshape(NW * ew_lg // D, D)
    m = _tc_max(lgf)
    exf = _tc_exp(lgf, m)
    ex_lay = exf.reshape(NW, ew_lg)

    k1_seg = _make_k1(ew_lg, ASEG, False)
    sparts = k1_seg(seg_sidx, ex_lay)
    sinv = _tc_sinv(sparts.reshape(NC, ASEG // D, D)).reshape(-1)

    ex_valid = ex_lay[:, :18750].reshape(-1).reshape(-1, H)
    k2_t3 = _make_k2(ec_t3, "attn")
    oh = []
    for h in range(H):
        cv = _lay_flat(ex_valid[:, h], NS, 18944, 0.0)
        ohf = jnp.concatenate(
            [k2_t3(vh[h], t3_gidx, t3_sidx[j], cv, t3_hidx[h],
                   sinv).reshape(SCR, D) for j in range(2)])
        oh.append(jnp.pad(ohf, ((0, NHP - 2 * SCR), (0, 0))))

    hp, s1h, s2h = _tc_attn_out(oh[0], oh[1], skip)
    bnp = params['bn']
    hfin = _tc_bn(hp, s1h, s2h, bnp['gamma'].reshape(1, -1),
                  bnp['beta'].reshape(1, -1))
    return hfin[:N], hfin[N:2 * N], hfin[2 * N:NH]
